# baseline (speedup 1.0000x reference)
import jax
import jax.numpy as jnp
from jax.experimental import pallas as pl
from jax.experimental.pallas import tpu as pltpu


def _bdot(a, b):
    return jnp.dot(a.astype(jnp.bfloat16), b.astype(jnp.bfloat16),
                   preferred_element_type=jnp.float32)


def _an_block(adj_ref, dinvc_ref, dinvr_ref):
    return ((adj_ref[...] * dinvc_ref[...]) * dinvr_ref[...]).astype(
        jnp.bfloat16)


def _pass_a(adj_ref, feats_ref, w1_ref, dinv_ref, v1_ref):
    deg = jnp.sum(adj_ref[...], axis=1)
    dinv = jax.lax.rsqrt(deg + 1e-9)
    dinv_ref[...] = dinv[:, None]
    v1_ref[...] = _bdot(feats_ref[...], w1_ref[...]).astype(jnp.bfloat16)


def _pass_b(adj_ref, dinvc_ref, dinvr_ref, v1_ref, b1_ref, w2_ref, v2_ref):
    an = _an_block(adj_ref, dinvc_ref, dinvr_ref)
    t = jnp.dot(an, v1_ref[...], preferred_element_type=jnp.float32)
    h = jnp.maximum(t + b1_ref[...], 0.0)
    v2_ref[...] = _bdot(h, w2_ref[...]).astype(jnp.bfloat16)


def _pass_c(adj_ref, dinvc_ref, dinvr_ref, v2_ref, b2_ref, wp_ref, bp_ref,
            out_ref):
    an = _an_block(adj_ref, dinvc_ref, dinvr_ref)
    t = jnp.dot(an, v2_ref[...], preferred_element_type=jnp.float32)
    h = jnp.maximum(t + b2_ref[...], 0.0)
    out_ref[...] = _bdot(h, wp_ref[...]) + bp_ref[...]


def kernel(feats, adj, W1, b1, W2, b2, Wp, bp):
    n, d = feats.shape
    h = W1.shape[1]
    bi = 400

    b1r = b1.reshape(1, h)
    b2r = b2.reshape(1, h)
    bpr = bp.reshape(1, 1)

    full = lambda *shape: pl.BlockSpec(shape, lambda i: (0,) * len(shape))
    rows = lambda *shape: pl.BlockSpec(shape, lambda i: (i,) + (0,) * (len(shape) - 1))

    params = pltpu.CompilerParams(dimension_semantics=("parallel",))

    dinv, v1 = pl.pallas_call(
        _pass_a,
        grid=(n // bi,),
        in_specs=[rows(bi, n), rows(bi, d), full(d, h)],
        out_specs=[rows(bi, 1), rows(bi, h)],
        out_shape=[
            jax.ShapeDtypeStruct((n, 1), jnp.float32),
            jax.ShapeDtypeStruct((n, h), jnp.bfloat16),
        ],
        compiler_params=params,
    )(adj, feats, W1)

    dinv_row = dinv.reshape(1, n)

    v2 = pl.pallas_call(
        _pass_b,
        grid=(n // bi,),
        in_specs=[rows(bi, n), rows(bi, 1), full(1, n), full(n, h),
                  full(1, h), full(h, h)],
        out_specs=rows(bi, h),
        out_shape=jax.ShapeDtypeStruct((n, h), jnp.bfloat16),
        compiler_params=params,
    )(adj, dinv, dinv_row, v1, b1r, W2.astype(jnp.bfloat16))

    out = pl.pallas_call(
        _pass_c,
        grid=(n // bi,),
        in_specs=[rows(bi, n), rows(bi, 1), full(1, n), full(n, h),
                  full(1, h), full(h, 1), full(1, 1)],
        out_specs=rows(bi, 1),
        out_shape=jax.ShapeDtypeStruct((n, 1), jnp.float32),
        compiler_params=params,
    )(adj, dinv, dinv_row, v2, b2r, Wp, bpr)

    return out


# speedup vs baseline: 1.0103x; 1.0103x over previous
import jax
import jax.numpy as jnp
from jax.experimental import pallas as pl
from jax.experimental.pallas import tpu as pltpu


def _bdot(a, b):
    return jnp.dot(a.astype(jnp.bfloat16), b.astype(jnp.bfloat16),
                   preferred_element_type=jnp.float32)


def _an_block(adj_ref, dinvc_ref, dinvr_ref):
    return ((adj_ref[...] * dinvc_ref[...]) * dinvr_ref[...]).astype(
        jnp.bfloat16)


def _make_ab(nb, bi):
    def _pass_ab(adj_ref, feats_ref, w1_ref, b1_ref, w2_ref,
                 dinv_ref, v2_ref, dinvc_ref, dinvr_ref, v1_ref):
        i = pl.program_id(0)

        @pl.when(i < nb)
        def _():
            deg = jnp.sum(adj_ref[...], axis=1)
            dinv = jax.lax.rsqrt(deg + 1e-9)
            dinv_ref[...] = dinv[:, None]
            dinvc_ref[pl.ds(i * bi, bi), :] = dinv[:, None]
            v1_ref[pl.ds(i * bi, bi), :] = _bdot(
                feats_ref[...], w1_ref[...]).astype(jnp.bfloat16)

        @pl.when(i == nb)
        def _():
            dinvr_ref[...] = dinvc_ref[...][:, 0][None, :]

        @pl.when(i >= nb)
        def _():
            k = i - nb
            an = ((adj_ref[...] * dinvc_ref[pl.ds(k * bi, bi), :])
                  * dinvr_ref[...]).astype(jnp.bfloat16)
            t = jnp.dot(an, v1_ref[...], preferred_element_type=jnp.float32)
            h = jnp.maximum(t + b1_ref[...], 0.0)
            v2_ref[...] = _bdot(h, w2_ref[...]).astype(jnp.bfloat16)

    return _pass_ab


def _pass_c(adj_ref, dinvc_ref, dinvr_ref, v2_ref, b2_ref, wp_ref, bp_ref,
            out_ref):
    an = _an_block(adj_ref, dinvc_ref, dinvr_ref)
    t = jnp.dot(an, v2_ref[...], preferred_element_type=jnp.float32)
    h = jnp.maximum(t + b2_ref[...], 0.0)
    out_ref[...] = _bdot(h, wp_ref[...]) + bp_ref[...]


def kernel(feats, adj, W1, b1, W2, b2, Wp, bp):
    n, d = feats.shape
    h = W1.shape[1]
    bi = 400

    b1r = b1.reshape(1, h)
    b2r = b2.reshape(1, h)
    bpr = bp.reshape(1, 1)

    full = lambda *shape: pl.BlockSpec(shape, lambda i: (0,) * len(shape))
    rows = lambda *shape: pl.BlockSpec(shape, lambda i: (i,) + (0,) * (len(shape) - 1))

    params = pltpu.CompilerParams(dimension_semantics=("parallel",))
    nb = n // bi
    cyc = lambda *shape: pl.BlockSpec(
        shape, lambda i: (jax.lax.rem(i, nb),) + (0,) * (len(shape) - 1))

    dinv, v2 = pl.pallas_call(
        _make_ab(nb, bi),
        grid=(2 * nb,),
        in_specs=[cyc(bi, n),
                  pl.BlockSpec((bi, d), lambda i: (jnp.minimum(i, nb - 1), 0)),
                  full(d, h), full(1, h), full(h, h)],
        out_specs=[
            pl.BlockSpec((bi, 1), lambda i: (jnp.minimum(i, nb - 1), 0)),
            pl.BlockSpec((bi, h), lambda i: (jnp.maximum(i - nb, 0), 0)),
        ],
        out_shape=[
            jax.ShapeDtypeStruct((n, 1), jnp.float32),
            jax.ShapeDtypeStruct((n, h), jnp.bfloat16),
        ],
        scratch_shapes=[
            pltpu.VMEM((n, 1), jnp.float32),
            pltpu.VMEM((1, n), jnp.float32),
            pltpu.VMEM((n, h), jnp.bfloat16),
        ],
        compiler_params=pltpu.CompilerParams(
            dimension_semantics=("arbitrary",),
            vmem_limit_bytes=63 * 1024 * 1024),
    )(adj, feats, W1, b1r, W2.astype(jnp.bfloat16))

    dinv_row = dinv.reshape(1, n)

    out = pl.pallas_call(
        _pass_c,
        grid=(n // bi,),
        in_specs=[rows(bi, n), rows(bi, 1), full(1, n), full(n, h),
                  full(1, h), full(h, 1), full(1, 1)],
        out_specs=rows(bi, 1),
        out_shape=jax.ShapeDtypeStruct((n, 1), jnp.float32),
        compiler_params=params,
    )(adj, dinv, dinv_row, v2, b2r, Wp, bpr)

    return out
